# baseline (speedup 1.0000x reference)
import functools
import math

import jax
import jax.numpy as jnp
from jax import lax
from jax.experimental import pallas as pl
from jax.experimental.pallas import tpu as pltpu
from jax.experimental.pallas import tpu_sc as plsc

_LANES = 16
_CHUNK = 80
_NGBUF = 4
_BLOCK = 2 * _CHUNK
_NSBUF = 2


@functools.cache
def _make_gather(B, V, D):
  info = plsc.get_sparse_core_info()
  nw = info.num_cores * info.num_subcores
  assert B % nw == 0
  b_per_w = B // nw
  assert b_per_w % (_NGBUF * _CHUNK) == 0
  n_chunks = b_per_w // _CHUNK
  n_outer = n_chunks // _NGBUF
  scale = math.sqrt(float(D))
  mesh = plsc.VectorSubcoreMesh(core_axis_name="c", subcore_axis_name="s")

  @functools.partial(
      pl.kernel,
      mesh=mesh,
      out_type=jax.ShapeDtypeStruct((B, D), jnp.float32),
      scratch_types=[
          pltpu.VMEM((b_per_w,), jnp.int32),
      ]
      + [pltpu.VMEM((_CHUNK, D), jnp.float32)] * _NGBUF
      + [pltpu.VMEM((_BLOCK, D), jnp.float32)] * _NSBUF
      + [pltpu.SemaphoreType.DMA] * (_NGBUF + _NSBUF),
  )
  def gather_kernel(table_hbm, idx_hbm, out_hbm, idx_v, *refs):
    gbuf = refs[:_NGBUF]
    sbuf = refs[_NGBUF:_NGBUF + _NSBUF]
    gsem = refs[_NGBUF + _NSBUF:2 * _NGBUF + _NSBUF]
    ssem = refs[2 * _NGBUF + _NSBUF:]
    wid = lax.axis_index("s") * info.num_cores + lax.axis_index("c")
    base = wid * b_per_w
    pltpu.sync_copy(idx_hbm.at[pl.ds(base, b_per_w)], idx_v)

    def gather_start(k, g):
      pltpu.async_copy(
          table_hbm.at[idx_v.at[pl.ds(k * _CHUNK, _CHUNK)]], gbuf[g], gsem[g]
      )

    def gather_wait(g):
      pltpu.make_async_copy(
          table_hbm.at[idx_v.at[pl.ds(0, _CHUNK)]], gbuf[g], gsem[g]
      ).wait()

    def scatter_start(bl, sb):
      pltpu.async_copy(
          sbuf[sb], out_hbm.at[pl.ds(base + bl * _BLOCK, _BLOCK)], ssem[sb]
      )

    def scatter_wait(sb):
      pltpu.make_async_copy(
          sbuf[sb], out_hbm.at[pl.ds(base, _BLOCK)], ssem[sb]
      ).wait()

    def scale_copy(g, sb, half):
      src = gbuf[g]
      dst = sbuf[sb]

      def row_body(r, carry):
        for j in range(D // _LANES):
          sl = pl.ds(j * _LANES, _LANES)
          dst[half * _CHUNK + r, sl] = src[r, sl] * scale
        return carry

      lax.fori_loop(0, _CHUNK, row_body, 0)

    for k in range(_NGBUF - 1):
      gather_start(k, k)

    def outer(i, carry):
      for m in range(_NGBUF):
        g = m
        k_dyn = i * _NGBUF + m
        gather_wait(g)

        nxt = (m + _NGBUF - 1) % _NGBUF
        if m == 0:
          gather_start(k_dyn + _NGBUF - 1, nxt)
        else:
          @pl.when(i < n_outer - 1)
          def _():
            gather_start(k_dyn + _NGBUF - 1, nxt)

        sb = (m // 2) % _NSBUF
        if m % 2 == 0:
          @pl.when(i > 0)
          def _():
            scatter_wait(sb)

        scale_copy(g, sb, m % 2)

        if m % 2 == 1:
          bl = i * 2 + m // 2
          scatter_start(bl, sb)
      return carry

    lax.fori_loop(0, n_outer, outer, 0)
    for sb in range(_NSBUF):
      scatter_wait(sb)

  return gather_kernel


def kernel(x, input_embedding):
  B1, B2 = x.shape
  V, D = input_embedding.shape
  idx = x.reshape(B1 * B2).astype(jnp.int32)
  out = _make_gather(B1 * B2, V, D)(input_embedding, idx)
  return out.reshape(B1, B2, D)


# speedup vs baseline: 1.0201x; 1.0201x over previous
import functools
import math

import jax
import jax.numpy as jnp
from jax import lax
from jax.experimental import pallas as pl
from jax.experimental.pallas import tpu as pltpu
from jax.experimental.pallas import tpu_sc as plsc

_LANES = 16
_CHUNK = 80
_NBUF = 10


@functools.cache
def _make_gather(B, V, D):
  info = plsc.get_sparse_core_info()
  nw = info.num_cores * info.num_subcores
  assert B % nw == 0
  b_per_w = B // nw
  assert b_per_w % (_NBUF * _CHUNK) == 0
  n_chunks = b_per_w // _CHUNK
  n_outer = n_chunks // _NBUF
  scale = math.sqrt(float(D))
  mesh = plsc.VectorSubcoreMesh(core_axis_name="c", subcore_axis_name="s")

  @functools.partial(
      pl.kernel,
      mesh=mesh,
      out_type=jax.ShapeDtypeStruct((B, D), jnp.float32),
      scratch_types=[
          pltpu.VMEM((b_per_w,), jnp.int32),
      ]
      + [pltpu.VMEM((_CHUNK, D), jnp.float32)] * _NBUF
      + [pltpu.SemaphoreType.DMA] * (2 * _NBUF),
  )
  def gather_kernel(table_hbm, idx_hbm, out_hbm, idx_v, *bufs_and_sems):
    rows = bufs_and_sems[:_NBUF]
    gsem = bufs_and_sems[_NBUF:2 * _NBUF]
    ssem = bufs_and_sems[2 * _NBUF:]
    wid = lax.axis_index("s") * info.num_cores + lax.axis_index("c")
    base = wid * b_per_w
    pltpu.sync_copy(idx_hbm.at[pl.ds(base, b_per_w)], idx_v)

    def gather_start(k, b):
      pltpu.async_copy(
          table_hbm.at[idx_v.at[pl.ds(k * _CHUNK, _CHUNK)]], rows[b], gsem[b]
      )

    def gather_wait(b):
      pltpu.make_async_copy(
          table_hbm.at[idx_v.at[pl.ds(0, _CHUNK)]], rows[b], gsem[b]
      ).wait()

    def scatter_start(k, b):
      pltpu.async_copy(
          rows[b], out_hbm.at[pl.ds(base + k * _CHUNK, _CHUNK)], ssem[b]
      )

    def scatter_wait(b):
      pltpu.make_async_copy(
          rows[b], out_hbm.at[pl.ds(base, _CHUNK)], ssem[b]
      ).wait()

    def do_scale(b):
      buf = rows[b]

      def row_body(i, carry):
        for j in range(D // _LANES):
          sl = pl.ds(j * _LANES, _LANES)
          buf[i, sl] = buf[i, sl] * scale
        return carry

      lax.fori_loop(0, _CHUNK, row_body, 0)

    for k in range(_NBUF - 1):
      gather_start(k, k)

    def outer(i, carry):
      for b in range(_NBUF):
        k = i * _NBUF + b
        nxt = (b + _NBUF - 1) % _NBUF
        gather_wait(b)
        if b == 0:
          @pl.when(i > 0)
          def _():
            scatter_wait(nxt)

          gather_start_i = i * _NBUF + _NBUF - 1
          pltpu.async_copy(
              table_hbm.at[idx_v.at[pl.ds(gather_start_i * _CHUNK, _CHUNK)]],
              rows[nxt],
              gsem[nxt],
          )
        else:
          @pl.when(i < n_outer - 1)
          def _():
            scatter_wait(nxt)
            pltpu.async_copy(
                table_hbm.at[
                    idx_v.at[pl.ds((i * _NBUF + b + _NBUF - 1) * _CHUNK,
                                   _CHUNK)]
                ],
                rows[nxt],
                gsem[nxt],
            )

        do_scale(b)
        scatter_start(k, b)
      return carry

    lax.fori_loop(0, n_outer, outer, 0)
    for b in range(_NBUF):
      scatter_wait(b)

  return gather_kernel


def kernel(x, input_embedding):
  B1, B2 = x.shape
  V, D = input_embedding.shape
  idx = x.reshape(B1 * B2).astype(jnp.int32)
  out = _make_gather(B1 * B2, V, D)(input_embedding, idx)
  return out.reshape(B1, B2, D)
